# baseline (speedup 1.0000x reference)
import functools
import jax
import jax.numpy as jnp
from jax.experimental import pallas as pl
from jax.experimental.pallas import tpu as pltpu

B, T, D = 64, 32, 2048
NRE = 7
DFF = 2048
BF = 512
KF = DFF // BF
GS = 12
MG = GS * T
NG = 11


def _moe_body(ge_ref, gact_ref, grow_ref, gval_ref, gw_ref,
              x_ref, w1_ref, b1_ref, w2_ref, b2_ref,
              out_ref, xg_ref, acc_ref):
    g = pl.program_id(0)
    kf = pl.program_id(1)

    @pl.when(gact_ref[g] > 0)
    def _():
        @pl.when(kf == 0)
        def _():
            for s in range(GS):
                xg_ref[s * T:(s + 1) * T] = x_ref[grow_ref[g, s] // 2]

        h = jnp.dot(xg_ref[...], w1_ref[0], preferred_element_type=jnp.float32)
        h = h + b1_ref[0, 0][None, :]
        gl = jax.nn.gelu(h)
        wcol = jnp.concatenate(
            [jnp.full((T, 1), gw_ref[g, s], jnp.float32) for s in range(GS)],
            axis=0)
        gl = gl * wcol
        contrib = jnp.dot(gl, w2_ref[0], preferred_element_type=jnp.float32)

        @pl.when(kf == 0)
        def _():
            acc_ref[...] = contrib

        @pl.when(kf > 0)
        def _():
            acc_ref[...] = acc_ref[...] + contrib

        @pl.when(kf == KF - 1)
        def _():
            total = acc_ref[...] + wcol * b2_ref[0, 0][None, :]
            for s in range(GS):
                @pl.when(gval_ref[g, s] > 0)
                def _():
                    out_ref[grow_ref[g, s]] = total[s * T:(s + 1) * T]


def _widx(kf, gact, g):
    return jnp.where(gact[g] > 0, kf, 0)


def _moe_ffn(ge, gact, grow, gval, gw, x, W1, b1r, W2, b2r):
    grid_spec = pltpu.PrefetchScalarGridSpec(
        num_scalar_prefetch=5,
        grid=(NG, KF),
        in_specs=[
            pl.BlockSpec((B, T, D),
                         lambda g, kf, ge, ga, gr, gv, gw: (0, 0, 0)),
            pl.BlockSpec((1, D, BF),
                         lambda g, kf, ge, ga, gr, gv, gw:
                         (ge[g], 0, _widx(kf, ga, g))),
            pl.BlockSpec((1, 1, BF),
                         lambda g, kf, ge, ga, gr, gv, gw:
                         (ge[g], 0, _widx(kf, ga, g))),
            pl.BlockSpec((1, BF, D),
                         lambda g, kf, ge, ga, gr, gv, gw:
                         (ge[g], _widx(kf, ga, g), 0)),
            pl.BlockSpec((1, 1, D),
                         lambda g, kf, ge, ga, gr, gv, gw: (ge[g], 0, 0)),
        ],
        out_specs=pl.BlockSpec((B, T, D),
                               lambda g, kf, ge, ga, gr, gv, gw: (0, 0, 0)),
        scratch_shapes=[
            pltpu.VMEM((MG, D), jnp.float32),
            pltpu.VMEM((MG, D), jnp.float32),
        ],
    )
    return pl.pallas_call(
        _moe_body,
        grid_spec=grid_spec,
        out_shape=jax.ShapeDtypeStruct((B, T, D), jnp.float32),
        compiler_params=pltpu.CompilerParams(
            dimension_semantics=("arbitrary", "arbitrary"),
            vmem_limit_bytes=67000000,
        ),
    )(ge, gact, grow, gval, gw, x, W1, b1r, W2, b2r)


@jax.jit
def kernel(x, Wg, W1, b1, W2, b2):
    x_avg = jnp.mean(x, axis=1)
    logits = x_avg @ Wg.T
    prob = jax.nn.softmax(logits, axis=-1)
    imp = jnp.sum(prob, axis=0)
    importance_loss = (jnp.std(imp, ddof=1) / jnp.mean(imp)) ** 2
    topv = jnp.max(prob, axis=-1)
    eid = jnp.argmax(prob, axis=-1).astype(jnp.int32)

    perm = jnp.argsort(eid, stable=True).astype(jnp.int32)
    counts = jnp.sum(eid[None, :] == jnp.arange(NRE, dtype=jnp.int32)[:, None],
                     axis=1).astype(jnp.int32)
    off = jnp.concatenate([jnp.zeros(1, jnp.int32), jnp.cumsum(counts)[:-1]])
    gpe = (counts + GS - 1) // GS
    gcum = jnp.cumsum(gpe)
    total_groups = gcum[-1]
    gids = jnp.arange(NG, dtype=jnp.int32)
    ge_raw = jnp.searchsorted(gcum, gids, side='right').astype(jnp.int32)
    valid_g = gids < total_groups
    gact = valid_g.astype(jnp.int32)
    ge = jnp.where(valid_g, ge_raw, NRE - 1).astype(jnp.int32)
    gi = gids - (gcum[ge] - gpe[ge])
    p0 = off[ge] + gi * GS
    pslots = p0[:, None] + jnp.arange(GS, dtype=jnp.int32)[None, :]
    gval = (pslots < (off[ge] + counts[ge])[:, None]) & valid_g[:, None]
    pclamp = jnp.minimum(pslots, B - 1)
    grow = perm[pclamp]
    gw = prob[grow // 2, ge[:, None]]
    gval = gval.astype(jnp.int32)

    b1r = b1.reshape(NRE, 1, DFF)
    b2r = b2.reshape(NRE, 1, D)
    output = _moe_ffn(ge, gact, grow, gval, gw, x, W1, b1r, W2, b2r)

    beam_scores = topv
    expert_route = eid[:, None]
    beam_idx = jnp.arange(B, dtype=jnp.int32)
    return (output, beam_scores, expert_route, beam_idx, importance_loss)


# speedup vs baseline: 1.1153x; 1.1153x over previous
import functools
import jax
import jax.numpy as jnp
from jax import lax
from jax.experimental import pallas as pl
from jax.experimental.pallas import tpu as pltpu
from jax.experimental.pallas import tpu_sc as plsc

B, T, D = 64, 32, 2048
NRE = 7
DFF = 2048
BF = 512
KF = DFF // BF
GS = 16
MG = GS * T
NG = 10
NCH = B // 16


def _gate_body(x_ref, wg_ref, lt_ref):
    xs = jnp.sum(x_ref[...], axis=1) * (1.0 / T)
    lt_ref[...] = lax.dot_general(
        wg_ref[...], xs, (((1,), (1,)), ((), ())),
        preferred_element_type=jnp.float32)


def _gate(x, Wg):
    return pl.pallas_call(
        _gate_body,
        out_shape=jax.ShapeDtypeStruct((NRE, B), jnp.float32),
    )(x, Wg)


def _route_body(lt_hbm,
                topv_hbm, eid_hbm, ge_hbm, gact_hbm, grow_hbm, gval_hbm,
                gw_hbm, loss_hbm,
                lt_v, prob_v, topv_v, eid_v, perm_v, rank_v, meta_v,
                grow_v, gval_v, gw_v, loss_v):
    wid = lax.axis_index("s") * 2 + lax.axis_index("c")
    if True:
        pltpu.sync_copy(lt_hbm, lt_v)
        glane = jnp.arange(16, dtype=jnp.int32)

        imp = [jnp.float32(0.0)] * NRE
        for c in range(NCH):
            sl = pl.ds(c * 16, 16)
            v = [lt_v[e, sl] for e in range(NRE)]
            m = v[0]
            for e in range(1, NRE):
                m = jnp.maximum(m, v[e])
            ex = [jnp.exp(v[e] - m) for e in range(NRE)]
            ssum = ex[0]
            for e in range(1, NRE):
                ssum = ssum + ex[e]
            p = [ex[e] / ssum for e in range(NRE)]
            best = p[0]
            bi = jnp.zeros((16,), jnp.int32)
            for e in range(1, NRE):
                upd = p[e] > best
                best = jnp.where(upd, p[e], best)
                bi = jnp.where(upd, jnp.int32(e), bi)
            topv_v[sl] = best
            eid_v[sl] = bi
            for e in range(NRE):
                prob_v[e, sl] = p[e]
                imp[e] = imp[e] + jnp.sum(p[e])

        cnt = [jnp.int32(0)] * NRE
        for c in range(NCH):
            sl = pl.ds(c * 16, 16)
            bi = eid_v[sl]
            rnk = jnp.zeros((16,), jnp.int32)
            for e in range(NRE):
                mi = (bi == e).astype(jnp.int32)
                pre = plsc.cumsum(mi) - mi
                rnk = rnk + mi * cnt[e] + mi * pre
                cnt[e] = cnt[e] + jnp.sum(mi)
            rank_v[sl] = rnk
        off = [jnp.int32(0)] * NRE
        for e in range(1, NRE):
            off[e] = off[e - 1] + cnt[e - 1]
        for c in range(NCH):
            sl = pl.ds(c * 16, 16)
            bi = eid_v[sl]
            base = jnp.zeros((16,), jnp.int32)
            for e in range(NRE):
                base = base + (bi == e).astype(jnp.int32) * off[e]
            rowid = glane + jnp.int32(c * 16)
            plsc.store_scatter(perm_v, [base + rank_v[sl]], rowid)

        gpe = [lax.shift_right_logical(cnt[e] + GS - 1, 4) for e in range(NRE)]
        gcum = [jnp.int32(0)] * NRE
        gcum[0] = gpe[0]
        for e in range(1, NRE):
            gcum[e] = gcum[e - 1] + gpe[e]
        total = gcum[NRE - 1]
        ge_vec = jnp.zeros((16,), jnp.int32)
        for e in range(NRE):
            ge_vec = ge_vec + (glane >= gcum[e]).astype(jnp.int32)
        gact_vec = (glane < total).astype(jnp.int32)
        ge_vec = jnp.where(gact_vec > 0, ge_vec, jnp.int32(NRE - 1))
        gstart = jnp.zeros((16,), jnp.int32)
        offsel = jnp.zeros((16,), jnp.int32)
        cntsel = jnp.zeros((16,), jnp.int32)
        for e in range(NRE):
            me = (ge_vec == e).astype(jnp.int32)
            gstart = gstart + me * (gcum[e] - gpe[e])
            offsel = offsel + me * off[e]
            cntsel = cntsel + me * cnt[e]
        p0_vec = offsel + (glane - gstart) * GS
        lim_vec = offsel + cntsel
        meta_v[0] = ge_vec
        meta_v[1] = gact_vec

        for g in range(NG):
            ge_g = ge_vec[g]
            act_g = gact_vec[g]
            p0_g = p0_vec[g]
            lim_g = lim_vec[g]
            idx = glane + p0_g
            idxc = jnp.minimum(idx, jnp.int32(B - 1))
            rows = plsc.load_gather(perm_v, [idxc])
            grow_v[g] = rows
            gval_v[g] = ((idx < lim_g) & (act_g > 0)).astype(jnp.int32)
            eidx = jnp.full((16,), ge_g, jnp.int32)
            rh = lax.shift_right_logical(rows, 1)
            gw_v[g] = plsc.load_gather(prob_v, [eidx, rh])

        tot = imp[0]
        for e in range(1, NRE):
            tot = tot + imp[e]
        mean = tot * jnp.float32(1.0 / NRE)
        var = jnp.float32(0.0)
        for e in range(NRE):
            dev = imp[e] - mean
            var = var + dev * dev
        var = var * jnp.float32(1.0 / (NRE - 1))
        loss_v[...] = (jnp.full((16,), var, jnp.float32)
                       / jnp.full((16,), mean * mean, jnp.float32))

        @pl.when(wid == 0)
        def _():
            pltpu.sync_copy(topv_v, topv_hbm)
            pltpu.sync_copy(eid_v, eid_hbm)
            pltpu.sync_copy(grow_v, grow_hbm)
            pltpu.sync_copy(gval_v, gval_hbm)
            pltpu.sync_copy(gw_v, gw_hbm)
            pltpu.sync_copy(loss_v, loss_hbm)
            pltpu.sync_copy(meta_v.at[0], ge_hbm)
            pltpu.sync_copy(meta_v.at[1], gact_hbm)


def _route(lt):
    mesh = plsc.VectorSubcoreMesh(core_axis_name="c", subcore_axis_name="s")
    f = pl.kernel(
        _route_body,
        mesh=mesh,
        compiler_params=pltpu.CompilerParams(needs_layout_passes=False),
        out_type=[
            jax.ShapeDtypeStruct((B,), jnp.float32),
            jax.ShapeDtypeStruct((B,), jnp.int32),
            jax.ShapeDtypeStruct((16,), jnp.int32),
            jax.ShapeDtypeStruct((16,), jnp.int32),
            jax.ShapeDtypeStruct((NG, GS), jnp.int32),
            jax.ShapeDtypeStruct((NG, GS), jnp.int32),
            jax.ShapeDtypeStruct((NG, GS), jnp.float32),
            jax.ShapeDtypeStruct((16,), jnp.float32),
        ],
        scratch_types=[
            pltpu.VMEM((NRE, B), jnp.float32),
            pltpu.VMEM((NRE, B), jnp.float32),
            pltpu.VMEM((B,), jnp.float32),
            pltpu.VMEM((B,), jnp.int32),
            pltpu.VMEM((B,), jnp.int32),
            pltpu.VMEM((B,), jnp.int32),
            pltpu.VMEM((2, 16), jnp.int32),
            pltpu.VMEM((NG, GS), jnp.int32),
            pltpu.VMEM((NG, GS), jnp.int32),
            pltpu.VMEM((NG, GS), jnp.float32),
            pltpu.VMEM((16,), jnp.float32),
        ],
    )
    return f(lt)


def _moe_body(ge_ref, gact_ref, grow_ref, gval_ref, gw_ref,
              x_ref, w1_ref, b1_ref, w2_ref, b2_ref,
              out_ref, xg_ref, acc_ref):
    g = pl.program_id(0)
    kf = pl.program_id(1)

    @pl.when(gact_ref[g] > 0)
    def _():
        @pl.when(kf == 0)
        def _():
            for s in range(GS):
                xg_ref[s * T:(s + 1) * T] = x_ref[grow_ref[g, s] // 2]

        h = jnp.dot(xg_ref[...], w1_ref[0], preferred_element_type=jnp.float32)
        h = h + b1_ref[0, 0][None, :]
        gl = jax.nn.gelu(h)
        wcol = jnp.concatenate(
            [jnp.full((T, 1), gw_ref[g, s], jnp.float32) for s in range(GS)],
            axis=0)
        gl = gl * wcol
        contrib = jnp.dot(gl, w2_ref[0], preferred_element_type=jnp.float32)

        @pl.when(kf == 0)
        def _():
            acc_ref[...] = contrib

        @pl.when(kf > 0)
        def _():
            acc_ref[...] = acc_ref[...] + contrib

        @pl.when(kf == KF - 1)
        def _():
            total = acc_ref[...] + wcol * b2_ref[0, 0][None, :]
            for s in range(GS):
                @pl.when(gval_ref[g, s] > 0)
                def _():
                    out_ref[grow_ref[g, s]] = total[s * T:(s + 1) * T]


def _widx(kf, gact, g):
    return jnp.where(gact[g] > 0, kf, 0)


def _moe_ffn(ge, gact, grow, gval, gw, x, W1, b1r, W2, b2r):
    grid_spec = pltpu.PrefetchScalarGridSpec(
        num_scalar_prefetch=5,
        grid=(NG, KF),
        in_specs=[
            pl.BlockSpec((B, T, D),
                         lambda g, kf, ge, ga, gr, gv, gw: (0, 0, 0)),
            pl.BlockSpec((1, D, BF),
                         lambda g, kf, ge, ga, gr, gv, gw:
                         (ge[g], 0, _widx(kf, ga, g))),
            pl.BlockSpec((1, 1, BF),
                         lambda g, kf, ge, ga, gr, gv, gw:
                         (ge[g], 0, _widx(kf, ga, g))),
            pl.BlockSpec((1, BF, D),
                         lambda g, kf, ge, ga, gr, gv, gw:
                         (ge[g], _widx(kf, ga, g), 0)),
            pl.BlockSpec((1, 1, D),
                         lambda g, kf, ge, ga, gr, gv, gw: (ge[g], 0, 0)),
        ],
        out_specs=pl.BlockSpec((B, T, D),
                               lambda g, kf, ge, ga, gr, gv, gw: (0, 0, 0)),
        scratch_shapes=[
            pltpu.VMEM((MG, D), jnp.float32),
            pltpu.VMEM((MG, D), jnp.float32),
        ],
    )
    return pl.pallas_call(
        _moe_body,
        grid_spec=grid_spec,
        out_shape=jax.ShapeDtypeStruct((B, T, D), jnp.float32),
        compiler_params=pltpu.CompilerParams(
            dimension_semantics=("arbitrary", "arbitrary"),
            vmem_limit_bytes=67000000,
        ),
    )(ge, gact, grow, gval, gw, x, W1, b1r, W2, b2r)


@jax.jit
def kernel(x, Wg, W1, b1, W2, b2):
    lt = _gate(x, Wg)
    topv, eid, ge16, gact16, grow, gval, gw, loss16 = _route(lt)
    ge = ge16[:NG]
    gact = gact16[:NG]

    b1r = b1.reshape(NRE, 1, DFF)
    b2r = b2.reshape(NRE, 1, D)
    output = _moe_ffn(ge, gact, grow, gval, gw, x, W1, b1r, W2, b2r)

    beam_scores = topv
    expert_route = eid[:, None]
    beam_idx = jnp.arange(B, dtype=jnp.int32)
    importance_loss = loss16[0]
    return (output, beam_scores, expert_route, beam_idx, importance_loss)
